# baseline (speedup 1.0000x reference)
import math

import jax
import jax.numpy as jnp
from jax.experimental import pallas as pl

N_HEADS = 16
D_MODEL = 1024
DK = D_MODEL // N_HEADS


def _to_heads(x):
    bm = x.shape[0]
    return x.reshape(bm, N_HEADS, DK).transpose(1, 0, 2)


def _proj_kernel(x_q, x_k, x_v, wq, wk, wv, bq, bk, bv, oq, ok, ov):
    dn = (((1,), (1,)), ((), ()))
    oq[:] = _to_heads(
        (jax.lax.dot_general(x_q[:], wq[:], dn,
                             preferred_element_type=jnp.float32)
         + bq[:]).astype(jnp.bfloat16))
    ok[:] = _to_heads(
        (jax.lax.dot_general(x_k[:], wk[:], dn,
                             preferred_element_type=jnp.float32)
         + bk[:]).astype(jnp.bfloat16))
    ov[:] = _to_heads(
        (jax.lax.dot_general(x_v[:], wv[:], dn,
                             preferred_element_type=jnp.float32)
         + bv[:]).astype(jnp.bfloat16))


def _attn_kernel(q_ref, k_ref, v_ref, o_ref):
    q = q_ref[0]
    k = k_ref[0]
    s = jax.lax.dot_general(q, k, (((1,), (1,)), ((), ())),
                            preferred_element_type=jnp.float32)
    s = s * (1.0 / math.sqrt(DK))
    m = jnp.max(s, axis=-1, keepdims=True)
    p = jnp.exp(s - m)
    l = jnp.sum(p, axis=-1, keepdims=True)
    ctx = jnp.dot(p.astype(jnp.bfloat16), v_ref[0],
                  preferred_element_type=jnp.float32)
    o_ref[0] = (ctx / l).astype(jnp.bfloat16)


def _oproj_kernel(x_ref, wo_ref, bo_ref, o_ref):
    bm = x_ref.shape[1]
    x = x_ref[:].transpose(1, 0, 2).reshape(bm, D_MODEL)
    dn = (((1,), (1,)), ((), ()))
    o_ref[:] = jax.lax.dot_general(x, wo_ref[:], dn,
                                   preferred_element_type=jnp.float32) + bo_ref[:]


def kernel(Q, K, V, Wq, bq, Wk, bk, Wv, bv, Wo, bo):
    B, L, D = Q.shape
    H, dk = N_HEADS, DK
    bf = jnp.bfloat16
    x_q = Q.reshape(L, D).astype(bf)
    x_k = K.reshape(L, D).astype(bf)
    x_v = V.reshape(L, D).astype(bf)
    bq2 = bq.reshape(1, D)
    bk2 = bk.reshape(1, D)
    bv2 = bv.reshape(1, D)
    bo2 = bo.reshape(1, D)

    BM = 512
    n_rb = L // BM

    w_spec = pl.BlockSpec((D, D), lambda i: (0, 0))
    b_spec = pl.BlockSpec((1, D), lambda i: (0, 0))
    row_spec = pl.BlockSpec((BM, D), lambda i: (i, 0))
    heads_spec = pl.BlockSpec((H, BM, dk), lambda i: (0, i, 0))

    qp, kp, vp = pl.pallas_call(
        _proj_kernel,
        grid=(n_rb,),
        in_specs=[row_spec, row_spec, row_spec,
                  w_spec, w_spec, w_spec,
                  b_spec, b_spec, b_spec],
        out_specs=[heads_spec, heads_spec, heads_spec],
        out_shape=[jax.ShapeDtypeStruct((H, L, dk), bf)] * 3,
    )(x_q, x_k, x_v, Wq.astype(bf), Wk.astype(bf), Wv.astype(bf),
      bq2, bk2, bv2)

    BQ = 512
    n_qb = L // BQ
    ctx = pl.pallas_call(
        _attn_kernel,
        grid=(H, n_qb),
        in_specs=[
            pl.BlockSpec((1, BQ, dk), lambda h, qb: (h, qb, 0)),
            pl.BlockSpec((1, L, dk), lambda h, qb: (h, 0, 0)),
            pl.BlockSpec((1, L, dk), lambda h, qb: (h, 0, 0)),
        ],
        out_specs=pl.BlockSpec((1, BQ, dk), lambda h, qb: (h, qb, 0)),
        out_shape=jax.ShapeDtypeStruct((H, L, dk), bf),
    )(qp, kp, vp)

    out = pl.pallas_call(
        _oproj_kernel,
        grid=(n_rb,),
        in_specs=[heads_spec, w_spec, b_spec],
        out_specs=row_spec,
        out_shape=jax.ShapeDtypeStruct((L, D), jnp.float32),
    )(ctx, Wo.astype(bf), bo2)

    return out.reshape(B, L, D)


# speedup vs baseline: 1.1326x; 1.1326x over previous
import math

import jax
import jax.numpy as jnp
from jax.experimental import pallas as pl

N_HEADS = 16
D_MODEL = 1024
DK = D_MODEL // N_HEADS
DV = 2 * DK


def _to_heads(x):
    bm = x.shape[0]
    return x.reshape(bm, N_HEADS, DK).transpose(1, 0, 2)


def _proj_kernel(x_q, x_k, x_v, wq, wk, wv, bq, bk, bv, oq, ok, ov):
    dn = (((1,), (1,)), ((), ()))
    scale = 1.0 / math.sqrt(DK)
    oq[:] = _to_heads(
        ((jax.lax.dot_general(x_q[:], wq[:], dn,
                              preferred_element_type=jnp.float32)
          + bq[:]) * scale).astype(jnp.bfloat16))
    ok[:] = _to_heads(
        (jax.lax.dot_general(x_k[:], wk[:], dn,
                             preferred_element_type=jnp.float32)
         + bk[:]).astype(jnp.bfloat16))
    vh = _to_heads(
        (jax.lax.dot_general(x_v[:], wv[:], dn,
                             preferred_element_type=jnp.float32)
         + bv[:]).astype(jnp.bfloat16))
    lane = jax.lax.broadcasted_iota(jnp.int32, vh.shape, 2)
    ones_col = jnp.where(lane == 0, 1.0, 0.0).astype(jnp.bfloat16)
    ov[:] = jnp.concatenate([vh, ones_col], axis=-1)


def _attn_kernel(q_ref, k_ref, v_ref, o_ref):
    q = q_ref[0]
    k = k_ref[0]
    s = jax.lax.dot_general(q, k, (((1,), (1,)), ((), ())),
                            preferred_element_type=jnp.float32)
    m = jnp.max(s, axis=-1, keepdims=True)
    p = jnp.exp(s - m).astype(jnp.bfloat16)
    o_ref[0] = jnp.dot(p, v_ref[0],
                       preferred_element_type=jnp.float32).astype(jnp.bfloat16)


def _oproj_kernel(x_ref, wo_ref, bo_ref, o_ref):
    bm = x_ref.shape[1]
    x = x_ref[:].astype(jnp.float32)
    ctx = x[:, :, :DK] / x[:, :, DK:DK + 1]
    xh = ctx.astype(jnp.bfloat16).transpose(1, 0, 2).reshape(bm, D_MODEL)
    dn = (((1,), (1,)), ((), ()))
    o_ref[:] = jax.lax.dot_general(xh, wo_ref[:], dn,
                                   preferred_element_type=jnp.float32) + bo_ref[:]


def kernel(Q, K, V, Wq, bq, Wk, bk, Wv, bv, Wo, bo):
    B, L, D = Q.shape
    H, dk = N_HEADS, DK
    bf = jnp.bfloat16
    x_q = Q.reshape(L, D).astype(bf)
    x_k = K.reshape(L, D).astype(bf)
    x_v = V.reshape(L, D).astype(bf)
    bq2 = bq.reshape(1, D)
    bk2 = bk.reshape(1, D)
    bv2 = bv.reshape(1, D)
    bo2 = bo.reshape(1, D)

    BM = 512
    n_rb = L // BM

    w_spec = pl.BlockSpec((D, D), lambda i: (0, 0))
    b_spec = pl.BlockSpec((1, D), lambda i: (0, 0))
    row_spec = pl.BlockSpec((BM, D), lambda i: (i, 0))
    heads_spec = pl.BlockSpec((H, BM, dk), lambda i: (0, i, 0))
    headsv_spec = pl.BlockSpec((H, BM, DV), lambda i: (0, i, 0))

    qp, kp, vp = pl.pallas_call(
        _proj_kernel,
        grid=(n_rb,),
        in_specs=[row_spec, row_spec, row_spec,
                  w_spec, w_spec, w_spec,
                  b_spec, b_spec, b_spec],
        out_specs=[heads_spec, heads_spec, headsv_spec],
        out_shape=[jax.ShapeDtypeStruct((H, L, dk), bf),
                   jax.ShapeDtypeStruct((H, L, dk), bf),
                   jax.ShapeDtypeStruct((H, L, DV), bf)],
    )(x_q, x_k, x_v, Wq.astype(bf), Wk.astype(bf), Wv.astype(bf),
      bq2, bk2, bv2)

    BQ = 512
    n_qb = L // BQ
    ctx = pl.pallas_call(
        _attn_kernel,
        grid=(H, n_qb),
        in_specs=[
            pl.BlockSpec((1, BQ, dk), lambda h, qb: (h, qb, 0)),
            pl.BlockSpec((1, L, dk), lambda h, qb: (h, 0, 0)),
            pl.BlockSpec((1, L, DV), lambda h, qb: (h, 0, 0)),
        ],
        out_specs=pl.BlockSpec((1, BQ, DV), lambda h, qb: (h, qb, 0)),
        out_shape=jax.ShapeDtypeStruct((H, L, DV), bf),
    )(qp, kp, vp)

    out = pl.pallas_call(
        _oproj_kernel,
        grid=(n_rb,),
        in_specs=[headsv_spec, w_spec, b_spec],
        out_specs=row_spec,
        out_shape=jax.ShapeDtypeStruct((L, D), jnp.float32),
    )(ctx, Wo.astype(bf), bo2)

    return out.reshape(B, L, D)


# speedup vs baseline: 1.4618x; 1.2907x over previous
import math

import jax
import jax.numpy as jnp
from jax.experimental import pallas as pl

N_HEADS = 16
D_MODEL = 1024
DK = D_MODEL // N_HEADS
DV = DK + 8


def _proj_kernel(x_q, x_k, x_v, wq, wk, wv, bq, bk, bv, oq, ok, ov):
    dn = (((2,), (1,)), ((), ()))
    scale = 1.0 / math.sqrt(DK)
    oq[:] = ((jax.lax.dot_general(wq[:], x_q[:], dn,
                                  preferred_element_type=jnp.float32)
              + bq[:]) * scale).astype(jnp.bfloat16)
    ok[:] = (jax.lax.dot_general(wk[:], x_k[:], dn,
                                 preferred_element_type=jnp.float32)
             + bk[:]).astype(jnp.bfloat16)
    vh = (jax.lax.dot_general(wv[:], x_v[:], dn,
                              preferred_element_type=jnp.float32)
          + bv[:]).astype(jnp.bfloat16)
    pad_shape = (N_HEADS, DV - DK, vh.shape[2])
    row = jax.lax.broadcasted_iota(jnp.int32, pad_shape, 1)
    ones_row = jnp.where(row == 0, 1.0, 0.0).astype(jnp.bfloat16)
    ov[:] = jnp.concatenate([vh, ones_row], axis=1)


def _attn_kernel(q_ref, k_ref, v_ref, o_ref):
    q = q_ref[0]
    k = k_ref[0]
    st = jax.lax.dot_general(k, q, (((0,), (0,)), ((), ())),
                             preferred_element_type=jnp.float32)
    m = jnp.max(st, axis=0, keepdims=True)
    p = jnp.exp(st - m).astype(jnp.bfloat16)
    o_ref[0] = jax.lax.dot_general(
        v_ref[0], p, (((1,), (0,)), ((), ())),
        preferred_element_type=jnp.float32).astype(jnp.bfloat16)


def _oproj_kernel(x_ref, wo_ref, bo_ref, o_ref):
    bm = x_ref.shape[2]
    x = x_ref[:].astype(jnp.float32)
    ctx = x[:, :DK, :] / x[:, DK:DK + 1, :]
    ctxn = ctx.astype(jnp.bfloat16).reshape(D_MODEL, bm)
    o_ref[:] = jax.lax.dot_general(ctxn, wo_ref[:], (((0,), (1,)), ((), ())),
                                   preferred_element_type=jnp.float32) + bo_ref[:]


def kernel(Q, K, V, Wq, bq, Wk, bk, Wv, bv, Wo, bo):
    B, L, D = Q.shape
    H, dk = N_HEADS, DK
    bf = jnp.bfloat16
    x_q = Q.reshape(L, D).astype(bf)
    x_k = K.reshape(L, D).astype(bf)
    x_v = V.reshape(L, D).astype(bf)
    wq3 = Wq.reshape(H, dk, D).astype(bf)
    wk3 = Wk.reshape(H, dk, D).astype(bf)
    wv3 = Wv.reshape(H, dk, D).astype(bf)
    bq3 = bq.reshape(H, dk, 1)
    bk3 = bk.reshape(H, dk, 1)
    bv3 = bv.reshape(H, dk, 1)
    bo2 = bo.reshape(1, D)

    BM = 512
    n_rb = L // BM

    w3_spec = pl.BlockSpec((H, dk, D), lambda i: (0, 0, 0))
    b3_spec = pl.BlockSpec((H, dk, 1), lambda i: (0, 0, 0))
    row_spec = pl.BlockSpec((BM, D), lambda i: (i, 0))
    headsT_spec = pl.BlockSpec((H, dk, BM), lambda i: (0, 0, i))
    headsTv_spec = pl.BlockSpec((H, DV, BM), lambda i: (0, 0, i))

    qp, kp, vp = pl.pallas_call(
        _proj_kernel,
        grid=(n_rb,),
        in_specs=[row_spec, row_spec, row_spec,
                  w3_spec, w3_spec, w3_spec,
                  b3_spec, b3_spec, b3_spec],
        out_specs=[headsT_spec, headsT_spec, headsTv_spec],
        out_shape=[jax.ShapeDtypeStruct((H, dk, L), bf),
                   jax.ShapeDtypeStruct((H, dk, L), bf),
                   jax.ShapeDtypeStruct((H, DV, L), bf)],
    )(x_q, x_k, x_v, wq3, wk3, wv3, bq3, bk3, bv3)

    BQ = 512
    n_qb = L // BQ
    ctx = pl.pallas_call(
        _attn_kernel,
        grid=(H, n_qb),
        in_specs=[
            pl.BlockSpec((1, dk, BQ), lambda h, qb: (h, 0, qb)),
            pl.BlockSpec((1, dk, L), lambda h, qb: (h, 0, 0)),
            pl.BlockSpec((1, DV, L), lambda h, qb: (h, 0, 0)),
        ],
        out_specs=pl.BlockSpec((1, DV, BQ), lambda h, qb: (h, 0, qb)),
        out_shape=jax.ShapeDtypeStruct((H, DV, L), bf),
    )(qp, kp, vp)

    wo_spec = pl.BlockSpec((D, D), lambda i: (0, 0))
    b_spec = pl.BlockSpec((1, D), lambda i: (0, 0))
    out = pl.pallas_call(
        _oproj_kernel,
        grid=(n_rb,),
        in_specs=[headsTv_spec, wo_spec, b_spec],
        out_specs=row_spec,
        out_shape=jax.ShapeDtypeStruct((L, D), jnp.float32),
    )(ctx, Wo.astype(bf), bo2)

    return out.reshape(B, L, D)


# speedup vs baseline: 1.7823x; 1.2192x over previous
import math

import jax
import jax.numpy as jnp
from jax.experimental import pallas as pl

N_HEADS = 16
D_MODEL = 1024
DK = D_MODEL // N_HEADS
DV = DK + 8


def _proj_kernel(x_q, x_k, x_v, wq, wk, wv, bq, bk, bv, oq, ok, ov):
    dn = (((2,), (1,)), ((), ()))
    scale = 1.0 / math.sqrt(DK)
    oq[:] = ((jax.lax.dot_general(wq[:], x_q[:], dn,
                                  preferred_element_type=jnp.float32)
              + bq[:]) * scale).astype(jnp.bfloat16)
    ok[:] = (jax.lax.dot_general(wk[:], x_k[:], dn,
                                 preferred_element_type=jnp.float32)
             + bk[:]).astype(jnp.bfloat16)
    vh = (jax.lax.dot_general(wv[:], x_v[:], dn,
                              preferred_element_type=jnp.float32)
          + bv[:]).astype(jnp.bfloat16)
    pad_shape = (N_HEADS, DV - DK, vh.shape[2])
    row = jax.lax.broadcasted_iota(jnp.int32, pad_shape, 1)
    ones_row = jnp.where(row == 0, 1.0, 0.0).astype(jnp.bfloat16)
    ov[:] = jnp.concatenate([vh, ones_row], axis=1)


def _attn_kernel(q_ref, k_ref, v_ref, o_ref):
    q = q_ref[0]
    k = k_ref[0]
    st = jax.lax.dot_general(k, q, (((0,), (0,)), ((), ())),
                             preferred_element_type=jnp.float32)
    kf = k.astype(jnp.float32)
    qf = q.astype(jnp.float32)
    kn = jnp.sqrt(jnp.max(jnp.sum(kf * kf, axis=0)))
    qn = jnp.sqrt(jnp.sum(qf * qf, axis=0, keepdims=True))
    p = jnp.exp(st - qn * kn).astype(jnp.bfloat16)
    o_ref[0] = jax.lax.dot_general(
        v_ref[0], p, (((1,), (0,)), ((), ())),
        preferred_element_type=jnp.float32).astype(jnp.bfloat16)


def _oproj_kernel(x_ref, wo_ref, bo_ref, o_ref):
    bm = x_ref.shape[2]
    x = x_ref[:].astype(jnp.float32)
    ctx = x[:, :DK, :] / x[:, DK:DK + 1, :]
    ctxn = ctx.astype(jnp.bfloat16).reshape(D_MODEL, bm)
    o_ref[:] = jax.lax.dot_general(ctxn, wo_ref[:], (((0,), (1,)), ((), ())),
                                   preferred_element_type=jnp.float32) + bo_ref[:]


def kernel(Q, K, V, Wq, bq, Wk, bk, Wv, bv, Wo, bo):
    B, L, D = Q.shape
    H, dk = N_HEADS, DK
    bf = jnp.bfloat16
    x_q = Q.reshape(L, D).astype(bf)
    x_k = K.reshape(L, D).astype(bf)
    x_v = V.reshape(L, D).astype(bf)
    wq3 = Wq.reshape(H, dk, D).astype(bf)
    wk3 = Wk.reshape(H, dk, D).astype(bf)
    wv3 = Wv.reshape(H, dk, D).astype(bf)
    bq3 = bq.reshape(H, dk, 1)
    bk3 = bk.reshape(H, dk, 1)
    bv3 = bv.reshape(H, dk, 1)
    bo2 = bo.reshape(1, D)

    BM = 512
    n_rb = L // BM

    w3_spec = pl.BlockSpec((H, dk, D), lambda i: (0, 0, 0))
    b3_spec = pl.BlockSpec((H, dk, 1), lambda i: (0, 0, 0))
    row_spec = pl.BlockSpec((BM, D), lambda i: (i, 0))
    headsT_spec = pl.BlockSpec((H, dk, BM), lambda i: (0, 0, i))
    headsTv_spec = pl.BlockSpec((H, DV, BM), lambda i: (0, 0, i))

    qp, kp, vp = pl.pallas_call(
        _proj_kernel,
        grid=(n_rb,),
        in_specs=[row_spec, row_spec, row_spec,
                  w3_spec, w3_spec, w3_spec,
                  b3_spec, b3_spec, b3_spec],
        out_specs=[headsT_spec, headsT_spec, headsTv_spec],
        out_shape=[jax.ShapeDtypeStruct((H, dk, L), bf),
                   jax.ShapeDtypeStruct((H, dk, L), bf),
                   jax.ShapeDtypeStruct((H, DV, L), bf)],
    )(x_q, x_k, x_v, wq3, wk3, wv3, bq3, bk3, bv3)

    BQ = 512
    n_qb = L // BQ
    ctx = pl.pallas_call(
        _attn_kernel,
        grid=(H, n_qb),
        in_specs=[
            pl.BlockSpec((1, dk, BQ), lambda h, qb: (h, 0, qb)),
            pl.BlockSpec((1, dk, L), lambda h, qb: (h, 0, 0)),
            pl.BlockSpec((1, DV, L), lambda h, qb: (h, 0, 0)),
        ],
        out_specs=pl.BlockSpec((1, DV, BQ), lambda h, qb: (h, 0, qb)),
        out_shape=jax.ShapeDtypeStruct((H, DV, L), bf),
    )(qp, kp, vp)

    wo_spec = pl.BlockSpec((D, D), lambda i: (0, 0))
    b_spec = pl.BlockSpec((1, D), lambda i: (0, 0))
    out = pl.pallas_call(
        _oproj_kernel,
        grid=(n_rb,),
        in_specs=[headsTv_spec, wo_spec, b_spec],
        out_specs=row_spec,
        out_shape=jax.ShapeDtypeStruct((L, D), jnp.float32),
    )(ctx, Wo.astype(bf), bo2)

    return out.reshape(B, L, D)


# speedup vs baseline: 1.9894x; 1.1162x over previous
import math

import jax
import jax.numpy as jnp
from jax.experimental import pallas as pl

N_HEADS = 16
D_MODEL = 1024
DK = D_MODEL // N_HEADS
DV = DK + 8


def _proj_kernel(x_q, x_k, x_v, wq, wk, wv, bq, bk, bv, oq, ok, ov):
    dn = (((2,), (1,)), ((), ()))
    scale = 1.0 / math.sqrt(DK)
    oq[:] = ((jax.lax.dot_general(wq[:], x_q[:], dn,
                                  preferred_element_type=jnp.float32)
              + bq[:]) * scale).astype(jnp.bfloat16)
    ok[:] = (jax.lax.dot_general(wk[:], x_k[:], dn,
                                 preferred_element_type=jnp.float32)
             + bk[:]).astype(jnp.bfloat16)
    vh = (jax.lax.dot_general(wv[:], x_v[:], dn,
                              preferred_element_type=jnp.float32)
          + bv[:]).astype(jnp.bfloat16)
    pad_shape = (N_HEADS, DV - DK, vh.shape[2])
    row = jax.lax.broadcasted_iota(jnp.int32, pad_shape, 1)
    ones_row = jnp.where(row == 0, 1.0, 0.0).astype(jnp.bfloat16)
    ov[:] = jnp.concatenate([vh, ones_row], axis=1)


def _attn_kernel(q_ref, k_ref, v_ref, o_ref):
    q = q_ref[0]
    k = k_ref[0]
    st = jax.lax.dot_general(k, q, (((0,), (0,)), ((), ())),
                             preferred_element_type=jnp.float32)
    kf = k.astype(jnp.float32)
    qf = q.astype(jnp.float32)
    kn = jnp.sqrt(jnp.max(jnp.sum(kf * kf, axis=0)))
    qn = jnp.sqrt(jnp.sum(qf * qf, axis=0, keepdims=True))
    p = jnp.exp(st - qn * kn).astype(jnp.bfloat16)
    o_ref[0] = jax.lax.dot_general(
        v_ref[0], p, (((1,), (0,)), ((), ())),
        preferred_element_type=jnp.float32).astype(jnp.bfloat16)


def _oproj_kernel(x_ref, wo_ref, bo_ref, o_ref):
    bm = x_ref.shape[2]
    x = x_ref[:].astype(jnp.float32)
    ctx = x[:, :DK, :] / x[:, DK:DK + 1, :]
    ctxn = ctx.astype(jnp.bfloat16).reshape(D_MODEL, bm)
    o_ref[:] = jax.lax.dot_general(ctxn, wo_ref[:], (((0,), (1,)), ((), ())),
                                   preferred_element_type=jnp.float32) + bo_ref[:]


def kernel(Q, K, V, Wq, bq, Wk, bk, Wv, bv, Wo, bo):
    B, L, D = Q.shape
    H, dk = N_HEADS, DK
    bf = jnp.bfloat16
    x_q = Q.reshape(L, D).astype(bf)
    x_k = K.reshape(L, D).astype(bf)
    x_v = V.reshape(L, D).astype(bf)
    wq3 = Wq.reshape(H, dk, D).astype(bf)
    wk3 = Wk.reshape(H, dk, D).astype(bf)
    wv3 = Wv.reshape(H, dk, D).astype(bf)
    bq3 = bq.reshape(H, dk, 1)
    bk3 = bk.reshape(H, dk, 1)
    bv3 = bv.reshape(H, dk, 1)
    bo2 = bo.reshape(1, D)

    BM = 512
    n_rb = L // BM

    w3_spec = pl.BlockSpec((H, dk, D), lambda i: (0, 0, 0))
    b3_spec = pl.BlockSpec((H, dk, 1), lambda i: (0, 0, 0))
    row_spec = pl.BlockSpec((BM, D), lambda i: (i, 0))
    headsT_spec = pl.BlockSpec((H, dk, BM), lambda i: (0, 0, i))
    headsTv_spec = pl.BlockSpec((H, DV, BM), lambda i: (0, 0, i))

    qp, kp, vp = pl.pallas_call(
        _proj_kernel,
        grid=(n_rb,),
        in_specs=[row_spec, row_spec, row_spec,
                  w3_spec, w3_spec, w3_spec,
                  b3_spec, b3_spec, b3_spec],
        out_specs=[headsT_spec, headsT_spec, headsTv_spec],
        out_shape=[jax.ShapeDtypeStruct((H, dk, L), bf),
                   jax.ShapeDtypeStruct((H, dk, L), bf),
                   jax.ShapeDtypeStruct((H, DV, L), bf)],
    )(x_q, x_k, x_v, wq3, wk3, wv3, bq3, bk3, bv3)

    BQ = 1024
    n_qb = L // BQ
    ctx = pl.pallas_call(
        _attn_kernel,
        grid=(H, n_qb),
        in_specs=[
            pl.BlockSpec((1, dk, BQ), lambda h, qb: (h, 0, qb)),
            pl.BlockSpec((1, dk, L), lambda h, qb: (h, 0, 0)),
            pl.BlockSpec((1, DV, L), lambda h, qb: (h, 0, 0)),
        ],
        out_specs=pl.BlockSpec((1, DV, BQ), lambda h, qb: (h, 0, qb)),
        out_shape=jax.ShapeDtypeStruct((H, DV, L), bf),
    )(qp, kp, vp)

    wo_spec = pl.BlockSpec((D, D), lambda i: (0, 0))
    b_spec = pl.BlockSpec((1, D), lambda i: (0, 0))
    out = pl.pallas_call(
        _oproj_kernel,
        grid=(n_rb,),
        in_specs=[headsTv_spec, wo_spec, b_spec],
        out_specs=row_spec,
        out_shape=jax.ShapeDtypeStruct((L, D), jnp.float32),
    )(ctx, Wo.astype(bf), bo2)

    return out.reshape(B, L, D)


# speedup vs baseline: 2.0958x; 1.0535x over previous
import math

import jax
import jax.numpy as jnp
from jax.experimental import pallas as pl

N_HEADS = 16
D_MODEL = 1024
DK = D_MODEL // N_HEADS
DV = DK + 8


def _proj_kernel(x_q, x_k, x_v, wq, wk, wv, bq, bk, bv, oq, ok, ov):
    dn = (((2,), (1,)), ((), ()))
    scale = 1.0 / math.sqrt(DK)
    oq[:] = ((jax.lax.dot_general(wq[:], x_q[:], dn,
                                  preferred_element_type=jnp.float32)
              + bq[:]) * scale).astype(jnp.bfloat16)
    ok[:] = (jax.lax.dot_general(wk[:], x_k[:], dn,
                                 preferred_element_type=jnp.float32)
             + bk[:]).astype(jnp.bfloat16)
    vh = (jax.lax.dot_general(wv[:], x_v[:], dn,
                              preferred_element_type=jnp.float32)
          + bv[:]).astype(jnp.bfloat16)
    pad_shape = (N_HEADS, DV - DK, vh.shape[2])
    row = jax.lax.broadcasted_iota(jnp.int32, pad_shape, 1)
    ones_row = jnp.where(row == 0, 1.0, 0.0).astype(jnp.bfloat16)
    ov[:] = jnp.concatenate([vh, ones_row], axis=1)


def _attn_kernel(q_ref, k_ref, v_ref, o_ref):
    q = q_ref[0]
    k = k_ref[0]
    st = jax.lax.dot_general(k, q, (((0,), (0,)), ((), ())),
                             preferred_element_type=jnp.float32)
    kf = k.astype(jnp.float32)
    qf = q.astype(jnp.float32)
    kn = jnp.sqrt(jnp.max(jnp.sum(kf * kf, axis=0)))
    qn = jnp.sqrt(jnp.sum(qf * qf, axis=0, keepdims=True))
    p = jnp.exp(st - qn * kn).astype(jnp.bfloat16)
    o_ref[0] = jax.lax.dot_general(
        v_ref[0], p, (((1,), (0,)), ((), ())),
        preferred_element_type=jnp.float32).astype(jnp.bfloat16)


def _oproj_kernel(x_ref, wo_ref, bo_ref, o_ref):
    bm = x_ref.shape[2]
    x = x_ref[:].astype(jnp.float32)
    ctx = x[:, :DK, :] / x[:, DK:DK + 1, :]
    ctxn = ctx.astype(jnp.bfloat16).reshape(D_MODEL, bm)
    o_ref[:] = jax.lax.dot_general(ctxn, wo_ref[:], (((0,), (1,)), ((), ())),
                                   preferred_element_type=jnp.float32) + bo_ref[:]


def kernel(Q, K, V, Wq, bq, Wk, bk, Wv, bv, Wo, bo):
    B, L, D = Q.shape
    H, dk = N_HEADS, DK
    bf = jnp.bfloat16
    x_q = Q.reshape(L, D).astype(bf)
    x_k = K.reshape(L, D).astype(bf)
    x_v = V.reshape(L, D).astype(bf)
    wq3 = Wq.reshape(H, dk, D).astype(bf)
    wk3 = Wk.reshape(H, dk, D).astype(bf)
    wv3 = Wv.reshape(H, dk, D).astype(bf)
    bq3 = bq.reshape(H, dk, 1)
    bk3 = bk.reshape(H, dk, 1)
    bv3 = bv.reshape(H, dk, 1)
    bo2 = bo.reshape(1, D)

    BM = 512
    n_rb = L // BM

    w3_spec = pl.BlockSpec((H, dk, D), lambda i: (0, 0, 0))
    b3_spec = pl.BlockSpec((H, dk, 1), lambda i: (0, 0, 0))
    row_spec = pl.BlockSpec((BM, D), lambda i: (i, 0))
    headsT_spec = pl.BlockSpec((H, dk, BM), lambda i: (0, 0, i))
    headsTv_spec = pl.BlockSpec((H, DV, BM), lambda i: (0, 0, i))

    qp, kp, vp = pl.pallas_call(
        _proj_kernel,
        grid=(n_rb,),
        in_specs=[row_spec, row_spec, row_spec,
                  w3_spec, w3_spec, w3_spec,
                  b3_spec, b3_spec, b3_spec],
        out_specs=[headsT_spec, headsT_spec, headsTv_spec],
        out_shape=[jax.ShapeDtypeStruct((H, dk, L), bf),
                   jax.ShapeDtypeStruct((H, dk, L), bf),
                   jax.ShapeDtypeStruct((H, DV, L), bf)],
    )(x_q, x_k, x_v, wq3, wk3, wv3, bq3, bk3, bv3)

    BQ = 2048
    n_qb = L // BQ
    ctx = pl.pallas_call(
        _attn_kernel,
        grid=(H, n_qb),
        in_specs=[
            pl.BlockSpec((1, dk, BQ), lambda h, qb: (h, 0, qb)),
            pl.BlockSpec((1, dk, L), lambda h, qb: (h, 0, 0)),
            pl.BlockSpec((1, DV, L), lambda h, qb: (h, 0, 0)),
        ],
        out_specs=pl.BlockSpec((1, DV, BQ), lambda h, qb: (h, 0, qb)),
        out_shape=jax.ShapeDtypeStruct((H, DV, L), bf),
    )(qp, kp, vp)

    wo_spec = pl.BlockSpec((D, D), lambda i: (0, 0))
    b_spec = pl.BlockSpec((1, D), lambda i: (0, 0))
    out = pl.pallas_call(
        _oproj_kernel,
        grid=(n_rb,),
        in_specs=[headsTv_spec, wo_spec, b_spec],
        out_specs=row_spec,
        out_shape=jax.ShapeDtypeStruct((L, D), jnp.float32),
    )(ctx, Wo.astype(bf), bo2)

    return out.reshape(B, L, D)


# speedup vs baseline: 2.3802x; 1.1357x over previous
import math

import jax
import jax.numpy as jnp
from jax.experimental import pallas as pl

N_HEADS = 16
D_MODEL = 1024
DK = D_MODEL // N_HEADS
DV = DK + 8


def _proj_kernel(x_q, x_k, x_v, wq, wk, wv, bq, bk, bv, oq, ok, ov):
    dn = (((2,), (1,)), ((), ()))
    scale = math.log2(math.e) / math.sqrt(DK)
    xq = x_q[:].astype(jnp.bfloat16)
    xk = x_k[:].astype(jnp.bfloat16)
    xv = x_v[:].astype(jnp.bfloat16)
    oq[:] = ((jax.lax.dot_general(wq[:], xq, dn,
                                  preferred_element_type=jnp.float32)
              + bq[:]) * scale).astype(jnp.bfloat16)
    kh = (jax.lax.dot_general(wk[:], xk, dn,
                              preferred_element_type=jnp.float32)
          + bk[:]).astype(jnp.bfloat16)
    vh = (jax.lax.dot_general(wv[:], xv, dn,
                              preferred_element_type=jnp.float32)
          + bv[:]).astype(jnp.bfloat16)
    pad_shape = (N_HEADS, DV - DK, vh.shape[2])
    row = jax.lax.broadcasted_iota(jnp.int32, pad_shape, 1)
    ones_row = jnp.where(row == 0, 1.0, 0.0).astype(jnp.bfloat16)
    ok[:] = jnp.concatenate([kh, ones_row], axis=1)
    ov[:] = jnp.concatenate([vh, ones_row], axis=1)


def _attn_kernel(q_ref, k_ref, v_ref, o_ref):
    q = q_ref[0]
    k = k_ref[0]
    qf = q.astype(jnp.float32)
    kf = k[:DK, :].astype(jnp.float32)
    kn = jnp.sqrt(jnp.max(jnp.sum(kf * kf, axis=0)))
    qn = jnp.sqrt(jnp.sum(qf * qf, axis=0, keepdims=True))
    shift = (-qn * kn).astype(jnp.bfloat16)
    pad = jnp.concatenate(
        [shift, jnp.zeros((DV - DK - 1, shift.shape[1]), jnp.bfloat16)], axis=0)
    q_aug = jnp.concatenate([q, pad], axis=0)
    st = jax.lax.dot_general(k, q_aug, (((0,), (0,)), ((), ())),
                             preferred_element_type=jnp.float32)
    p = jnp.exp2(st).astype(jnp.bfloat16)
    o_ref[0] = jax.lax.dot_general(
        v_ref[0], p, (((1,), (0,)), ((), ())),
        preferred_element_type=jnp.float32).astype(jnp.bfloat16)


def _oproj_kernel(x_ref, wo_ref, bo_ref, o_ref):
    bm = x_ref.shape[2]
    x = x_ref[:].astype(jnp.float32)
    ctx = x[:, :DK, :] / x[:, DK:DK + 1, :]
    ctxn = ctx.astype(jnp.bfloat16).reshape(D_MODEL, bm)
    o_ref[:] = jax.lax.dot_general(ctxn, wo_ref[:], (((0,), (1,)), ((), ())),
                                   preferred_element_type=jnp.float32) + bo_ref[:]


def kernel(Q, K, V, Wq, bq, Wk, bk, Wv, bv, Wo, bo):
    B, L, D = Q.shape
    H, dk = N_HEADS, DK
    bf = jnp.bfloat16
    x_q = Q.reshape(L, D)
    x_k = K.reshape(L, D)
    x_v = V.reshape(L, D)
    wq3 = Wq.reshape(H, dk, D).astype(bf)
    wk3 = Wk.reshape(H, dk, D).astype(bf)
    wv3 = Wv.reshape(H, dk, D).astype(bf)
    bq3 = bq.reshape(H, dk, 1)
    bk3 = bk.reshape(H, dk, 1)
    bv3 = bv.reshape(H, dk, 1)
    bo2 = bo.reshape(1, D)

    BM = 512
    n_rb = L // BM

    w3_spec = pl.BlockSpec((H, dk, D), lambda i: (0, 0, 0))
    b3_spec = pl.BlockSpec((H, dk, 1), lambda i: (0, 0, 0))
    row_spec = pl.BlockSpec((BM, D), lambda i: (i, 0))
    headsT_spec = pl.BlockSpec((H, dk, BM), lambda i: (0, 0, i))
    headsTv_spec = pl.BlockSpec((H, DV, BM), lambda i: (0, 0, i))

    qp, kp, vp = pl.pallas_call(
        _proj_kernel,
        grid=(n_rb,),
        in_specs=[row_spec, row_spec, row_spec,
                  w3_spec, w3_spec, w3_spec,
                  b3_spec, b3_spec, b3_spec],
        out_specs=[headsT_spec, headsTv_spec, headsTv_spec],
        out_shape=[jax.ShapeDtypeStruct((H, dk, L), bf),
                   jax.ShapeDtypeStruct((H, DV, L), bf),
                   jax.ShapeDtypeStruct((H, DV, L), bf)],
    )(x_q, x_k, x_v, wq3, wk3, wv3, bq3, bk3, bv3)

    BQ = 2048
    n_qb = L // BQ
    ctx = pl.pallas_call(
        _attn_kernel,
        grid=(H, n_qb),
        in_specs=[
            pl.BlockSpec((1, dk, BQ), lambda h, qb: (h, 0, qb)),
            pl.BlockSpec((1, DV, L), lambda h, qb: (h, 0, 0)),
            pl.BlockSpec((1, DV, L), lambda h, qb: (h, 0, 0)),
        ],
        out_specs=pl.BlockSpec((1, DV, BQ), lambda h, qb: (h, 0, qb)),
        out_shape=jax.ShapeDtypeStruct((H, DV, L), bf),
    )(qp, kp, vp)

    wo_spec = pl.BlockSpec((D, D), lambda i: (0, 0))
    b_spec = pl.BlockSpec((1, D), lambda i: (0, 0))
    out = pl.pallas_call(
        _oproj_kernel,
        grid=(n_rb,),
        in_specs=[headsTv_spec, wo_spec, b_spec],
        out_specs=row_spec,
        out_shape=jax.ShapeDtypeStruct((L, D), jnp.float32),
    )(ctx, Wo.astype(bf), bo2)

    return out.reshape(B, L, D)
